# baseline (speedup 1.0000x reference)
import functools

import jax
import jax.numpy as jnp
from jax import lax
from jax.experimental import pallas as pl
from jax.experimental.pallas import tpu as pltpu
from jax.experimental.pallas import tpu_sc as plsc

B = 64
KS = 32
VS = 32
M = 1_000_000
BLK = 8192
GRID = (M + BLK - 1) // BLK
EPS = 1e-8


def _scan_body(q_ref, keys_ref, idx_ref, bestv_ref, besti_ref):
    i = pl.program_id(0)

    @pl.when(i == 0)
    def _init():
        bestv_ref[...] = jnp.full((B, 1), -jnp.inf, dtype=jnp.float32)
        besti_ref[...] = jnp.zeros((B, 1), dtype=jnp.int32)

    q = q_ref[...]
    qn = q / jnp.maximum(jnp.sqrt(jnp.sum(q * q, axis=1, keepdims=True)), EPS)
    k = keys_ref[:, 0, :]
    kn = k / jnp.maximum(jnp.sqrt(jnp.sum(k * k, axis=1, keepdims=True)), EPS)
    sim = lax.dot_general(qn, kn, (((1,), (1,)), ((), ())),
                          preferred_element_type=jnp.float32,
                          precision=lax.Precision.HIGHEST)
    col = lax.broadcasted_iota(jnp.int32, (B, BLK), 1)
    valid = M - i * BLK
    sim = jnp.where(col < valid, sim, -jnp.inf)
    bmax = jnp.max(sim, axis=1, keepdims=True)
    cand = jnp.where(sim == bmax, col, jnp.int32(2**31 - 1))
    bidx = jnp.min(cand, axis=1, keepdims=True) + i * BLK
    better = bmax > bestv_ref[...]
    bestv_ref[...] = jnp.where(better, bmax, bestv_ref[...])
    besti_ref[...] = jnp.where(better, bidx, besti_ref[...])

    @pl.when(i == GRID - 1)
    def _finish():
        idx_ref[...] = besti_ref[...] * 2 + 1


_scan = pl.pallas_call(
    _scan_body,
    grid=(GRID,),
    in_specs=[
        pl.BlockSpec((B, KS), lambda i: (0, 0)),
        pl.BlockSpec((BLK, 2, KS), lambda i: (i, 0, 0)),
    ],
    out_specs=pl.BlockSpec((B, 1), lambda i: (0, 0)),
    out_shape=jax.ShapeDtypeStruct((B, 1), jnp.int32),
    scratch_shapes=[
        pltpu.VMEM((B, 1), jnp.float32),
        pltpu.VMEM((B, 1), jnp.int32),
    ],
)


NW_USED = 8
ROWS_PER_W = B // NW_USED


@functools.cache
def _make_sc_gather():
    mesh = plsc.VectorSubcoreMesh(core_axis_name="c", subcore_axis_name="s")

    @functools.partial(
        pl.kernel,
        mesh=mesh,
        compiler_params=pltpu.CompilerParams(use_tc_tiling_on_sc=False),
        out_type=jax.ShapeDtypeStruct((B, VS), jnp.float32),
        scratch_types=[
            pltpu.VMEM((ROWS_PER_W,), jnp.int32),
            pltpu.VMEM((ROWS_PER_W, VS), jnp.float32),
            pltpu.SemaphoreType.DMA,
        ],
    )
    def gather_k(mem2_hbm, idx_hbm, out_hbm, idx_v, rows_v, sem):
        c = lax.axis_index("c")
        s = lax.axis_index("s")
        wid = s * 2 + c

        @pl.when(wid < NW_USED)
        def _():
            base = wid * ROWS_PER_W
            pltpu.sync_copy(idx_hbm.at[pl.ds(base, ROWS_PER_W)], idx_v)
            pltpu.async_copy(mem2_hbm.at[idx_v], rows_v, sem).wait()
            pltpu.sync_copy(rows_v, out_hbm.at[pl.ds(base, ROWS_PER_W)])

    return gather_k


@jax.jit
def kernel(query, memory):
    mem3 = memory.reshape(M, 2, KS)
    idx2 = _scan(query, mem3)
    mem2 = memory.reshape(2 * M, KS)
    return _make_sc_gather()(mem2, idx2.reshape(B))


# speedup vs baseline: 2.1330x; 2.1330x over previous
import functools

import jax
import jax.numpy as jnp
from jax import lax
from jax.experimental import pallas as pl
from jax.experimental.pallas import tpu as pltpu
from jax.experimental.pallas import tpu_sc as plsc

B = 64
KS = 32
VS = 32
W = KS + VS
M = 1_000_000
F = 4
WF = F * W
MF = M // F
BLKF = 2000
GRID = MF // BLKF
EPS = 1e-8
BIG = 1 << 28


def _scan_body(qn_ref, mem_ref, idx_ref, bestv_ref, besti_ref):
    i = pl.program_id(0)

    @pl.when(i == 0)
    def _init():
        bestv_ref[...] = jnp.full((B, 1), -jnp.inf, dtype=jnp.float32)
        besti_ref[...] = jnp.zeros((B, 1), dtype=jnp.int32)

    qnb = qn_ref[...].astype(jnp.bfloat16)
    zb = jnp.zeros((B, KS), jnp.bfloat16)
    qq = jnp.concatenate(
        [jnp.concatenate([qnb if j == g else zb for j in range(F)], axis=1)
         for g in range(F)], axis=0)

    w = mem_ref[...]
    kn = []
    for g in range(F):
        keys_g = w[:, g * W:g * W + KS]
        ksq = jnp.sum(keys_g * keys_g, axis=1, keepdims=True)
        rcp = 1.0 / jnp.maximum(jnp.sqrt(ksq), EPS)
        kn.append((keys_g * rcp).astype(jnp.bfloat16))
    knb = jnp.concatenate(kn, axis=1)

    res = lax.dot_general(qq, knb, (((1,), (1,)), ((), ())),
                          preferred_element_type=jnp.float32)

    sims = [res[B * g:B * (g + 1), :] for g in range(F)]
    bm = jnp.full((B, 1), -jnp.inf, jnp.float32)
    for sim_g in sims:
        bm = jnp.maximum(bm, jnp.max(sim_g, axis=1, keepdims=True))

    iota = lax.broadcasted_iota(jnp.int32, (B, BLKF), 1)
    bidx = jnp.full((B, 1), 4 * BIG, jnp.int32)
    for g in range(F):
        cand = jnp.where(sims[g] == bm, iota, BIG)
        c_g = jnp.min(cand, axis=1, keepdims=True)
        bidx = jnp.minimum(bidx, 4 * (i * BLKF + c_g) + g)

    better = bm > bestv_ref[...]
    bestv_ref[...] = jnp.where(better, bm, bestv_ref[...])
    besti_ref[...] = jnp.where(better, bidx, besti_ref[...])

    @pl.when(i == GRID - 1)
    def _finish():
        idx_ref[...] = besti_ref[...]


_scan = pl.pallas_call(
    _scan_body,
    grid=(GRID,),
    in_specs=[
        pl.BlockSpec((B, KS), lambda i: (0, 0)),
        pl.BlockSpec((BLKF, WF), lambda i: (i, 0)),
    ],
    out_specs=pl.BlockSpec((B, 1), lambda i: (0, 0)),
    out_shape=jax.ShapeDtypeStruct((B, 1), jnp.int32),
    scratch_shapes=[
        pltpu.VMEM((B, 1), jnp.float32),
        pltpu.VMEM((B, 1), jnp.int32),
    ],
)

_NC = 2
_NW_USED = B // 8


@functools.cache
def _sc_gather_fn():
    mesh = plsc.VectorSubcoreMesh(core_axis_name="c", subcore_axis_name="s")

    @functools.partial(
        pl.kernel, mesh=mesh,
        out_type=jax.ShapeDtypeStruct((B, 2 * W), jnp.float32),
        scratch_types=[
            pltpu.VMEM((8,), jnp.int32),
            pltpu.VMEM((8, 2 * W), jnp.float32),
            pltpu.SemaphoreType.DMA,
        ],
    )
    def _sc_gather(table_hbm, idx_hbm, out_hbm, idx_v, rows_v, sem):
        wid = lax.axis_index("s") * _NC + lax.axis_index("c")

        @pl.when(wid < _NW_USED)
        def _():
            base = wid * 8
            pltpu.sync_copy(idx_hbm.at[pl.ds(base, 8)], idx_v)
            pltpu.async_copy(table_hbm.at[idx_v], rows_v, sem).wait()
            pltpu.sync_copy(rows_v, out_hbm.at[pl.ds(base, 8)])

    return _sc_gather


@jax.jit
def kernel(query, memory):
    q_norm = jnp.maximum(jnp.linalg.norm(query, axis=1, keepdims=True), EPS)
    qn = query / q_norm
    mem4 = memory.reshape(MF, WF)
    idx = _scan(qn, mem4)[:, 0]
    rows = _sc_gather_fn()(memory.reshape(M // 2, 2 * W), idx >> 1)
    odd = (idx & 1)[:, None] == 1
    return jnp.where(odd, rows[:, W + KS:], rows[:, KS:W])


# speedup vs baseline: 5.0859x; 2.3844x over previous
import jax
import jax.numpy as jnp
from jax import lax
from jax.experimental import pallas as pl
from jax.experimental.pallas import tpu as pltpu

B = 64
KS = 32
VS = 32
W = KS + VS
M = 1_000_000
BLK = 8000
GRID = M // BLK
EPS = 1e-8
SENT = 1 << 28


def _scan_body(qn_ref, mem_ref, idx_ref, bestv_ref, besti_ref):
    i = pl.program_id(0)

    @pl.when(i == 0)
    def _init():
        bestv_ref[...] = jnp.full((B, 1), -jnp.inf, dtype=jnp.float32)
        besti_ref[...] = jnp.zeros((B, 1), dtype=jnp.int32)

    qnb = qn_ref[...].astype(jnp.bfloat16)

    keys_t = jnp.transpose(mem_ref[:, :KS])
    ksq = jnp.sum(keys_t * keys_t, axis=0, keepdims=True)
    rcp = 1.0 / jnp.maximum(jnp.sqrt(ksq), EPS)
    kn_t = (keys_t * rcp).astype(jnp.bfloat16)

    sim = lax.dot_general(qnb, kn_t, (((1,), (0,)), ((), ())),
                          preferred_element_type=jnp.float32)

    bm = jnp.max(sim, axis=1, keepdims=True)
    iota = lax.broadcasted_iota(jnp.int32, (B, BLK), 1)
    cand = jnp.where(sim == bm, iota, SENT)
    c = jnp.min(cand, axis=1, keepdims=True)
    bidx = i * BLK + c

    better = bm > bestv_ref[...]
    bestv_ref[...] = jnp.where(better, bm, bestv_ref[...])
    besti_ref[...] = jnp.where(better, bidx, besti_ref[...])

    @pl.when(i == GRID - 1)
    def _finish():
        idx_ref[...] = besti_ref[...]


_scan = pl.pallas_call(
    _scan_body,
    grid=(GRID,),
    in_specs=[
        pl.BlockSpec((B, KS), lambda i: (0, 0)),
        pl.BlockSpec((BLK, W), lambda i: (i, 0)),
    ],
    out_specs=pl.BlockSpec((B, 1), lambda i: (0, 0)),
    out_shape=jax.ShapeDtypeStruct((B, 1), jnp.int32),
    scratch_shapes=[
        pltpu.VMEM((B, 1), jnp.float32),
        pltpu.VMEM((B, 1), jnp.int32),
    ],
)


def _gather_body(idx_ref, rows_ref, out_ref):
    i = pl.program_id(0)
    sub = idx_ref[i] % 8
    out_ref[pl.ds(i % 8, 1), :] = rows_ref[pl.ds(sub, 1), KS:]


_gather = pl.pallas_call(
    _gather_body,
    grid_spec=pltpu.PrefetchScalarGridSpec(
        num_scalar_prefetch=1,
        grid=(B,),
        in_specs=[pl.BlockSpec((8, W), lambda i, idx_ref: (idx_ref[i] // 8, 0))],
        out_specs=pl.BlockSpec((8, VS), lambda i, idx_ref: (i // 8, 0)),
    ),
    out_shape=jax.ShapeDtypeStruct((B, VS), jnp.float32),
)


@jax.jit
def kernel(query, memory):
    q_norm = jnp.maximum(jnp.linalg.norm(query, axis=1, keepdims=True), EPS)
    qn = query / q_norm
    idx = _scan(qn, memory)
    return _gather(idx[:, 0], memory)


# speedup vs baseline: 5.1460x; 1.0118x over previous
import jax
import jax.numpy as jnp
from jax import lax
from jax.experimental import pallas as pl
from jax.experimental.pallas import tpu as pltpu

B = 64
KS = 32
VS = 32
W = KS + VS
M = 1_000_000
BLK = 8000
GRID = M // BLK
EPS = 1e-8
SENT = 1 << 28


def _scan_body(qn_ref, mem_ref, idx_ref, bestv_ref, besti_ref):
    i = pl.program_id(0)

    @pl.when(i == 0)
    def _init():
        bestv_ref[...] = jnp.full((B, 1), -jnp.inf, dtype=jnp.float32)
        besti_ref[...] = jnp.zeros((B, 1), dtype=jnp.int32)

    qnb = qn_ref[...].astype(jnp.bfloat16)

    keys_t = jnp.transpose(mem_ref[:, :KS])
    ksq = jnp.sum(keys_t * keys_t, axis=0, keepdims=True)
    rcp = 1.0 / jnp.maximum(jnp.sqrt(ksq), EPS)
    kn_t = (keys_t * rcp).astype(jnp.bfloat16)

    sim = lax.dot_general(qnb, kn_t, (((1,), (0,)), ((), ())),
                          preferred_element_type=jnp.float32)

    bm = jnp.max(sim, axis=1, keepdims=True)
    c = jnp.argmax(sim, axis=1).reshape(B, 1).astype(jnp.int32)
    bidx = i * BLK + c

    better = bm > bestv_ref[...]
    bestv_ref[...] = jnp.where(better, bm, bestv_ref[...])
    besti_ref[...] = jnp.where(better, bidx, besti_ref[...])

    @pl.when(i == GRID - 1)
    def _finish():
        idx_ref[...] = besti_ref[...]


_scan = pl.pallas_call(
    _scan_body,
    grid=(GRID,),
    in_specs=[
        pl.BlockSpec((B, KS), lambda i: (0, 0)),
        pl.BlockSpec((BLK, W), lambda i: (i, 0)),
    ],
    out_specs=pl.BlockSpec((B, 1), lambda i: (0, 0)),
    out_shape=jax.ShapeDtypeStruct((B, 1), jnp.int32),
    scratch_shapes=[
        pltpu.VMEM((B, 1), jnp.float32),
        pltpu.VMEM((B, 1), jnp.int32),
    ],
)


def _gather_body(idx_ref, rows_ref, out_ref):
    i = pl.program_id(0)
    sub = idx_ref[i] % 8
    out_ref[pl.ds(i % 8, 1), :] = rows_ref[pl.ds(sub, 1), KS:]


_gather = pl.pallas_call(
    _gather_body,
    grid_spec=pltpu.PrefetchScalarGridSpec(
        num_scalar_prefetch=1,
        grid=(B,),
        in_specs=[pl.BlockSpec((8, W), lambda i, idx_ref: (idx_ref[i] // 8, 0))],
        out_specs=pl.BlockSpec((8, VS), lambda i, idx_ref: (i // 8, 0)),
    ),
    out_shape=jax.ShapeDtypeStruct((B, VS), jnp.float32),
)


@jax.jit
def kernel(query, memory):
    q_norm = jnp.maximum(jnp.linalg.norm(query, axis=1, keepdims=True), EPS)
    qn = query / q_norm
    idx = _scan(qn, memory)
    return _gather(idx[:, 0], memory)
